# baseline (speedup 1.0000x reference)
import functools

import jax
import jax.numpy as jnp
from jax import lax
from jax.experimental import pallas as pl
from jax.experimental.pallas import tpu as pltpu
from jax.experimental.pallas import tpu_sc as plsc

EMBED_DIM = 32
BATCH = 16384
FIELDS = 26
TOTAL = BATCH * FIELDS
NUM_CORES = 2
NUM_SUBCORES = 16
NW = NUM_CORES * NUM_SUBCORES
ROWS_PER_W = TOTAL // NW
GRP = 128
G_PER_CHUNK = 13
CHUNK = GRP * G_PER_CHUNK
GROUPS_PER_W = ROWS_PER_W // GRP
N_CHUNKS = GROUPS_PER_W // G_PER_CHUNK

N_PAIRS = FIELDS * (BATCH // GRP)
PAIRS_PER_W = N_PAIRS // NW
PAIR_WORDS = GRP * EMBED_DIM

_mesh = plsc.VectorSubcoreMesh(core_axis_name="c", subcore_axis_name="s")


@functools.partial(
    pl.kernel,
    mesh=_mesh,
    out_type=jax.ShapeDtypeStruct((TOTAL, EMBED_DIM), jnp.float32),
    scratch_types=[
        pltpu.VMEM((GROUPS_PER_W, GRP), jnp.int32),
        pltpu.VMEM((CHUNK, EMBED_DIM), jnp.float32),
        pltpu.VMEM((CHUNK, EMBED_DIM), jnp.float32),
        pltpu.SemaphoreType.DMA,
        pltpu.SemaphoreType.DMA,
    ],
    compiler_params=pltpu.CompilerParams(use_tc_tiling_on_sc=False),
)
def _sc_gather(idx_hbm, table_hbm, out_hbm, idx_v, rows0, rows1, sem0, sem1):
    wid = lax.axis_index("s") * NUM_CORES + lax.axis_index("c")
    grp0 = wid * GROUPS_PER_W
    row0 = grp0 * GRP

    pltpu.sync_copy(idx_hbm.at[pl.ds(grp0, GROUPS_PER_W)], idx_v)

    def fire(c, buf, sem):
        for g in range(G_PER_CHUNK):
            pltpu.async_copy(
                table_hbm.at[idx_v.at[c * G_PER_CHUNK + g]],
                buf.at[pl.ds(g * GRP, GRP)],
                sem,
            )

    def drain(buf, sem):
        pltpu.make_async_copy(out_hbm.at[pl.ds(0, CHUNK)], buf, sem).wait()

    def flush(c, buf):
        pltpu.sync_copy(buf, out_hbm.at[pl.ds(row0 + c * CHUNK, CHUNK)])

    fire(0, rows0, sem0)
    fire(1, rows1, sem1)

    def step(i, carry):
        c0 = 2 * i
        drain(rows0, sem0)
        flush(c0, rows0)

        @pl.when(i < N_CHUNKS // 2 - 1)
        def _():
            fire(c0 + 2, rows0, sem0)

        drain(rows1, sem1)
        flush(c0 + 1, rows1)

        @pl.when(i < N_CHUNKS // 2 - 1)
        def _():
            fire(c0 + 3, rows1, sem1)

        return carry

    lax.fori_loop(0, N_CHUNKS // 2, step, 0)


@functools.partial(
    pl.kernel,
    mesh=_mesh,
    out_type=jax.ShapeDtypeStruct((FIELDS, EMBED_DIM, BATCH), jnp.float32),
    scratch_types=[
        pltpu.VMEM((PAIR_WORDS,), jnp.float32),
        pltpu.VMEM((8, GRP), jnp.float32),
        pltpu.VMEM((8, GRP), jnp.float32),
        pltpu.SemaphoreType.DMA,
    ],
    compiler_params=pltpu.CompilerParams(
        use_tc_tiling_on_sc=True, needs_layout_passes=False
    ),
)
def _sc_out_xpose(x_hbm, out_hbm, in_v, tile0, tile1, sem):
    wid = lax.axis_index("s") * NUM_CORES + lax.axis_index("c")
    q0 = wid * PAIRS_PER_W
    lanes = lax.iota(jnp.int32, 16)

    def pair_body(t, carry):
        q = q0 + t
        f = q // (BATCH // GRP)
        jb = q % (BATCH // GRP)
        pltpu.sync_copy(x_hbm.at[pl.ds(q * PAIR_WORDS, PAIR_WORDS)], in_v)
        for i in range(4):
            tile = tile0 if i % 2 == 0 else tile1
            for dp in range(8):
                for k in range(8):
                    idx = (16 * k + lanes) * EMBED_DIM + (8 * i + dp)
                    tile[dp, pl.ds(16 * k, 16)] = plsc.load_gather(in_v, [idx])
            pltpu.sync_copy(
                tile, out_hbm.at[f, pl.ds(8 * i, 8), pl.ds(jb * GRP, GRP)]
            )
        return carry

    lax.fori_loop(0, PAIRS_PER_W, pair_body, 0)


def kernel(indices, table):
    fidx = indices.T.astype(jnp.int32).reshape(TOTAL // GRP, GRP)
    x = _sc_gather(fidx, table)
    out3 = _sc_out_xpose(x.reshape(TOTAL * EMBED_DIM))
    return jnp.transpose(out3, (2, 0, 1))


# speedup vs baseline: 1.0712x; 1.0712x over previous
import functools

import jax
import jax.numpy as jnp
from jax import lax
from jax.experimental import pallas as pl
from jax.experimental.pallas import tpu as pltpu
from jax.experimental.pallas import tpu_sc as plsc

EMBED_DIM = 32
BATCH = 16384
FIELDS = 26
TOTAL = BATCH * FIELDS
NUM_CORES = 2
NUM_SUBCORES = 16
NW = NUM_CORES * NUM_SUBCORES
ROWS_PER_W = TOTAL // NW
GRP = 128
G_PER_CHUNK = 13
CHUNK = GRP * G_PER_CHUNK
GROUPS_PER_W = ROWS_PER_W // GRP
N_CHUNKS = GROUPS_PER_W // G_PER_CHUNK

N_PAIRS = FIELDS * (BATCH // GRP)
PAIRS_PER_W = N_PAIRS // NW
PAIR_WORDS = GRP * EMBED_DIM

_mesh = plsc.VectorSubcoreMesh(core_axis_name="c", subcore_axis_name="s")


@functools.partial(
    pl.kernel,
    mesh=_mesh,
    out_type=jax.ShapeDtypeStruct((TOTAL, EMBED_DIM), jnp.float32),
    scratch_types=[
        pltpu.VMEM((GROUPS_PER_W, GRP), jnp.int32),
        pltpu.VMEM((CHUNK, EMBED_DIM), jnp.float32),
        pltpu.VMEM((CHUNK, EMBED_DIM), jnp.float32),
        pltpu.SemaphoreType.DMA,
        pltpu.SemaphoreType.DMA,
    ],
    compiler_params=pltpu.CompilerParams(use_tc_tiling_on_sc=False),
)
def _sc_gather(idx_hbm, table_hbm, out_hbm, idx_v, rows0, rows1, sem0, sem1):
    wid = lax.axis_index("s") * NUM_CORES + lax.axis_index("c")
    grp0 = wid * GROUPS_PER_W
    row0 = grp0 * GRP

    pltpu.sync_copy(idx_hbm.at[pl.ds(grp0, GROUPS_PER_W)], idx_v)

    def fire(c, buf, sem):
        for g in range(G_PER_CHUNK):
            pltpu.async_copy(
                table_hbm.at[idx_v.at[c * G_PER_CHUNK + g]],
                buf.at[pl.ds(g * GRP, GRP)],
                sem,
            )

    def drain(buf, sem):
        pltpu.make_async_copy(out_hbm.at[pl.ds(0, CHUNK)], buf, sem).wait()

    def flush(c, buf):
        pltpu.sync_copy(buf, out_hbm.at[pl.ds(row0 + c * CHUNK, CHUNK)])

    fire(0, rows0, sem0)
    fire(1, rows1, sem1)

    def step(i, carry):
        c0 = 2 * i
        drain(rows0, sem0)
        flush(c0, rows0)

        @pl.when(i < N_CHUNKS // 2 - 1)
        def _():
            fire(c0 + 2, rows0, sem0)

        drain(rows1, sem1)
        flush(c0 + 1, rows1)

        @pl.when(i < N_CHUNKS // 2 - 1)
        def _():
            fire(c0 + 3, rows1, sem1)

        return carry

    lax.fori_loop(0, N_CHUNKS // 2, step, 0)


@functools.partial(
    pl.kernel,
    mesh=_mesh,
    out_type=jax.ShapeDtypeStruct((FIELDS, EMBED_DIM, BATCH), jnp.float32),
    scratch_types=[
        pltpu.VMEM((PAIR_WORDS,), jnp.float32),
        pltpu.VMEM((PAIR_WORDS,), jnp.float32),
        pltpu.VMEM((EMBED_DIM, GRP), jnp.float32),
        pltpu.VMEM((EMBED_DIM, GRP), jnp.float32),
        pltpu.SemaphoreType.DMA,
        pltpu.SemaphoreType.DMA,
        pltpu.SemaphoreType.DMA,
        pltpu.SemaphoreType.DMA,
    ],
    compiler_params=pltpu.CompilerParams(
        use_tc_tiling_on_sc=True, needs_layout_passes=False
    ),
)
def _sc_out_xpose(
    x_hbm, out_hbm, in0, in1, ob0, ob1, isem0, isem1, osem0, osem1
):
    wid = lax.axis_index("s") * NUM_CORES + lax.axis_index("c")
    q0 = wid * PAIRS_PER_W
    lanes32 = lax.iota(jnp.int32, 16) * EMBED_DIM

    def start_in(t, buf, sem):
        pltpu.async_copy(
            x_hbm.at[pl.ds((q0 + t) * PAIR_WORDS, PAIR_WORDS)], buf, sem
        )

    def drain(buf, sem):
        pltpu.make_async_copy(x_hbm.at[pl.ds(0, PAIR_WORDS)], buf, sem).wait()

    def drain_ob(buf, sem):
        pltpu.make_async_copy(
            out_hbm.at[0, pl.ds(0, EMBED_DIM), pl.ds(0, GRP)], buf, sem
        ).wait()

    def xpose(src, dst):
        for i in range(4):
            for dp in range(8):
                for k in range(8):
                    idx = lanes32 + (16 * EMBED_DIM * k + 8 * i + dp)
                    dst[8 * i + dp, pl.ds(16 * k, 16)] = plsc.load_gather(
                        src, [idx]
                    )

    def write_out(t, buf, sem):
        q = q0 + t
        f = q // (BATCH // GRP)
        jb = q % (BATCH // GRP)
        pltpu.async_copy(
            buf,
            out_hbm.at[f, pl.ds(0, EMBED_DIM), pl.ds(jb * GRP, GRP)],
            sem,
        )

    HALF = PAIRS_PER_W // 2

    pltpu.sync_copy(x_hbm.at[pl.ds(q0 * PAIR_WORDS, PAIR_WORDS)], in0)
    start_in(1, in1, isem1)

    def step(u, carry):
        tA = 2 * u

        @pl.when(u > 0)
        def _():
            drain(in0, isem0)

        xpose(in0, ob0)

        @pl.when(u < HALF - 1)
        def _():
            start_in(tA + 2, in0, isem0)

        @pl.when(u > 0)
        def _():
            drain_ob(ob0, osem0)

        write_out(tA, ob0, osem0)

        drain(in1, isem1)
        xpose(in1, ob1)

        @pl.when(u < HALF - 1)
        def _():
            start_in(tA + 3, in1, isem1)

        @pl.when(u > 0)
        def _():
            drain_ob(ob1, osem1)

        write_out(tA + 1, ob1, osem1)
        return carry

    lax.fori_loop(0, HALF, step, 0)
    drain_ob(ob0, osem0)
    drain_ob(ob1, osem1)


def kernel(indices, table):
    fidx = indices.T.astype(jnp.int32).reshape(TOTAL // GRP, GRP)
    x = _sc_gather(fidx, table)
    out3 = _sc_out_xpose(x.reshape(TOTAL * EMBED_DIM))
    return jnp.transpose(out3, (2, 0, 1))


# speedup vs baseline: 1.2320x; 1.1501x over previous
import functools

import jax
import jax.numpy as jnp
from jax import lax
from jax.experimental import pallas as pl
from jax.experimental.pallas import tpu as pltpu
from jax.experimental.pallas import tpu_sc as plsc

EMBED_DIM = 32
BATCH = 16384
FIELDS = 26
TOTAL = BATCH * FIELDS
NUM_CORES = 2
NUM_SUBCORES = 16
NW = NUM_CORES * NUM_SUBCORES
ROWS_PER_W = TOTAL // NW
GRP = 128
G_PER_CHUNK = 13
CHUNK = GRP * G_PER_CHUNK
GROUPS_PER_W = ROWS_PER_W // GRP
N_CHUNKS = GROUPS_PER_W // G_PER_CHUNK

N_PAIRS = FIELDS * (BATCH // GRP)
PAIRS_PER_W = N_PAIRS // NW
PAIR_WORDS = GRP * EMBED_DIM

_mesh = plsc.VectorSubcoreMesh(core_axis_name="c", subcore_axis_name="s")


@functools.partial(
    pl.kernel,
    mesh=_mesh,
    out_type=jax.ShapeDtypeStruct((TOTAL, EMBED_DIM), jnp.float32),
    scratch_types=[
        pltpu.VMEM((GROUPS_PER_W, GRP), jnp.int32),
        pltpu.VMEM((CHUNK, EMBED_DIM), jnp.float32),
        pltpu.VMEM((CHUNK, EMBED_DIM), jnp.float32),
        pltpu.SemaphoreType.DMA,
        pltpu.SemaphoreType.DMA,
    ],
    compiler_params=pltpu.CompilerParams(use_tc_tiling_on_sc=False),
)
def _sc_gather(idx_hbm, table_hbm, out_hbm, idx_v, rows0, rows1, sem0, sem1):
    wid = lax.axis_index("s") * NUM_CORES + lax.axis_index("c")
    grp0 = wid * GROUPS_PER_W
    row0 = grp0 * GRP

    pltpu.sync_copy(idx_hbm.at[pl.ds(grp0, GROUPS_PER_W)], idx_v)

    def fire(c, buf, sem):
        for g in range(G_PER_CHUNK):
            pltpu.async_copy(
                table_hbm.at[idx_v.at[c * G_PER_CHUNK + g]],
                buf.at[pl.ds(g * GRP, GRP)],
                sem,
            )

    def drain(buf, sem):
        pltpu.make_async_copy(out_hbm.at[pl.ds(0, CHUNK)], buf, sem).wait()

    def flush(c, buf):
        pltpu.sync_copy(buf, out_hbm.at[pl.ds(row0 + c * CHUNK, CHUNK)])

    fire(0, rows0, sem0)
    fire(1, rows1, sem1)

    def step(i, carry):
        c0 = 2 * i
        drain(rows0, sem0)
        flush(c0, rows0)

        @pl.when(i < N_CHUNKS // 2 - 1)
        def _():
            fire(c0 + 2, rows0, sem0)

        drain(rows1, sem1)
        flush(c0 + 1, rows1)

        @pl.when(i < N_CHUNKS // 2 - 1)
        def _():
            fire(c0 + 3, rows1, sem1)

        return carry

    lax.fori_loop(0, N_CHUNKS // 2, step, 0)


@functools.partial(
    pl.kernel,
    mesh=_mesh,
    out_type=jax.ShapeDtypeStruct((FIELDS, EMBED_DIM, BATCH), jnp.float32),
    scratch_types=[
        pltpu.VMEM((PAIR_WORDS,), jnp.float32),
        pltpu.VMEM((PAIR_WORDS,), jnp.float32),
        pltpu.VMEM((EMBED_DIM, GRP), jnp.float32),
        pltpu.VMEM((EMBED_DIM, GRP), jnp.float32),
        pltpu.SemaphoreType.DMA,
        pltpu.SemaphoreType.DMA,
        pltpu.SemaphoreType.DMA,
        pltpu.SemaphoreType.DMA,
    ],
    compiler_params=pltpu.CompilerParams(
        use_tc_tiling_on_sc=True, needs_layout_passes=False
    ),
)
def _sc_out_xpose(
    x_hbm, out_hbm, in0, in1, ob0, ob1, isem0, isem1, osem0, osem1
):
    wid = lax.axis_index("s") * NUM_CORES + lax.axis_index("c")
    q0 = wid * PAIRS_PER_W
    lanes32 = lax.iota(jnp.int32, 16) * EMBED_DIM

    def start_in(t, buf, sem):
        pltpu.async_copy(
            x_hbm.at[pl.ds((q0 + t) * PAIR_WORDS, PAIR_WORDS)], buf, sem
        )

    def drain(buf, sem):
        pltpu.make_async_copy(x_hbm.at[pl.ds(0, PAIR_WORDS)], buf, sem).wait()

    def drain_ob(buf, sem):
        pltpu.make_async_copy(
            out_hbm.at[0, pl.ds(0, EMBED_DIM), pl.ds(0, GRP)], buf, sem
        ).wait()

    def xpose(src, dst):
        for i in range(4):
            for dp2 in range(4):
                vals = [
                    plsc.load_gather(
                        src,
                        [lanes32 + (16 * EMBED_DIM * k + 8 * i + 2 * dp2 + h)],
                    )
                    for h in range(2)
                    for k in range(8)
                ]
                for h in range(2):
                    for k in range(8):
                        dst[8 * i + 2 * dp2 + h, pl.ds(16 * k, 16)] = vals[
                            8 * h + k
                        ]

    def write_out(t, buf, sem):
        q = q0 + t
        f = q // (BATCH // GRP)
        jb = q % (BATCH // GRP)
        pltpu.async_copy(
            buf,
            out_hbm.at[f, pl.ds(0, EMBED_DIM), pl.ds(jb * GRP, GRP)],
            sem,
        )

    HALF = PAIRS_PER_W // 2

    pltpu.sync_copy(x_hbm.at[pl.ds(q0 * PAIR_WORDS, PAIR_WORDS)], in0)
    start_in(1, in1, isem1)

    def step(u, carry):
        tA = 2 * u

        @pl.when(u > 0)
        def _():
            drain(in0, isem0)

        xpose(in0, ob0)

        @pl.when(u < HALF - 1)
        def _():
            start_in(tA + 2, in0, isem0)

        @pl.when(u > 0)
        def _():
            drain_ob(ob0, osem0)

        write_out(tA, ob0, osem0)

        drain(in1, isem1)
        xpose(in1, ob1)

        @pl.when(u < HALF - 1)
        def _():
            start_in(tA + 3, in1, isem1)

        @pl.when(u > 0)
        def _():
            drain_ob(ob1, osem1)

        write_out(tA + 1, ob1, osem1)
        return carry

    lax.fori_loop(0, HALF, step, 0)
    drain_ob(ob0, osem0)
    drain_ob(ob1, osem1)


def kernel(indices, table):
    fidx = indices.T.astype(jnp.int32).reshape(TOTAL // GRP, GRP)
    x = _sc_gather(fidx, table)
    out3 = _sc_out_xpose(x.reshape(TOTAL * EMBED_DIM))
    return jnp.transpose(out3, (2, 0, 1))


# speedup vs baseline: 1.5990x; 1.2979x over previous
import functools

import jax
import jax.numpy as jnp
from jax import lax
from jax.experimental import pallas as pl
from jax.experimental.pallas import tpu as pltpu
from jax.experimental.pallas import tpu_sc as plsc

EMBED_DIM = 32
BATCH = 16384
FIELDS = 26
TOTAL = BATCH * FIELDS
NUM_CORES = 2
NUM_SUBCORES = 16
NW = NUM_CORES * NUM_SUBCORES
ROWS_PER_W = TOTAL // NW
GRP = 128
G_PER_CHUNK = 13
CHUNK = GRP * G_PER_CHUNK
GROUPS_PER_W = ROWS_PER_W // GRP
N_CHUNKS = GROUPS_PER_W // G_PER_CHUNK

N_PAIRS = FIELDS * (BATCH // GRP)
PAIRS_PER_W = N_PAIRS // NW
PAIR_WORDS = GRP * EMBED_DIM

_mesh = plsc.VectorSubcoreMesh(core_axis_name="c", subcore_axis_name="s")


@functools.partial(
    pl.kernel,
    mesh=_mesh,
    out_type=jax.ShapeDtypeStruct((TOTAL, EMBED_DIM), jnp.float32),
    scratch_types=[
        pltpu.VMEM((GROUPS_PER_W, GRP), jnp.int32),
        pltpu.VMEM((CHUNK, EMBED_DIM), jnp.float32),
        pltpu.VMEM((CHUNK, EMBED_DIM), jnp.float32),
        pltpu.SemaphoreType.DMA,
        pltpu.SemaphoreType.DMA,
    ],
    compiler_params=pltpu.CompilerParams(use_tc_tiling_on_sc=False),
)
def _sc_gather(idx_hbm, table_hbm, out_hbm, idx_v, rows0, rows1, sem0, sem1):
    wid = lax.axis_index("s") * NUM_CORES + lax.axis_index("c")
    grp0 = wid * GROUPS_PER_W
    row0 = grp0 * GRP

    pltpu.sync_copy(idx_hbm.at[pl.ds(grp0, GROUPS_PER_W)], idx_v)

    def fire(c, buf, sem):
        for g in range(G_PER_CHUNK):
            pltpu.async_copy(
                table_hbm.at[idx_v.at[c * G_PER_CHUNK + g]],
                buf.at[pl.ds(g * GRP, GRP)],
                sem,
            )

    def drain(buf, sem):
        pltpu.make_async_copy(out_hbm.at[pl.ds(0, CHUNK)], buf, sem).wait()

    def flush(c, buf):
        pltpu.sync_copy(buf, out_hbm.at[pl.ds(row0 + c * CHUNK, CHUNK)])

    fire(0, rows0, sem0)
    fire(1, rows1, sem1)

    def step(i, carry):
        c0 = 2 * i
        drain(rows0, sem0)
        flush(c0, rows0)

        @pl.when(i < N_CHUNKS // 2 - 1)
        def _():
            fire(c0 + 2, rows0, sem0)

        drain(rows1, sem1)
        flush(c0 + 1, rows1)

        @pl.when(i < N_CHUNKS // 2 - 1)
        def _():
            fire(c0 + 3, rows1, sem1)

        return carry

    lax.fori_loop(0, N_CHUNKS // 2, step, 0)


@functools.partial(
    pl.kernel,
    mesh=_mesh,
    out_type=jax.ShapeDtypeStruct((FIELDS, EMBED_DIM, BATCH), jnp.float32),
    scratch_types=[
        pltpu.VMEM((PAIR_WORDS,), jnp.float32),
        pltpu.VMEM((PAIR_WORDS,), jnp.float32),
        pltpu.VMEM((EMBED_DIM, GRP), jnp.float32),
        pltpu.VMEM((EMBED_DIM, GRP), jnp.float32),
        pltpu.SemaphoreType.DMA,
        pltpu.SemaphoreType.DMA,
        pltpu.SemaphoreType.DMA,
        pltpu.SemaphoreType.DMA,
    ],
    compiler_params=pltpu.CompilerParams(
        use_tc_tiling_on_sc=True, needs_layout_passes=False
    ),
)
def _sc_out_xpose(
    x_hbm, out_hbm, in0, in1, ob0, ob1, isem0, isem1, osem0, osem1
):
    wid = lax.axis_index("s") * NUM_CORES + lax.axis_index("c")
    q0 = wid * PAIRS_PER_W
    lanes32 = lax.iota(jnp.int32, 16) * EMBED_DIM

    def start_in(t, buf, sem):
        pltpu.async_copy(
            x_hbm.at[pl.ds((q0 + t) * PAIR_WORDS, PAIR_WORDS)], buf, sem
        )

    def drain(buf, sem):
        pltpu.make_async_copy(x_hbm.at[pl.ds(0, PAIR_WORDS)], buf, sem).wait()

    def drain_ob(buf, sem):
        pltpu.make_async_copy(
            out_hbm.at[0, pl.ds(0, EMBED_DIM), pl.ds(0, GRP)], buf, sem
        ).wait()

    lanes = lax.iota(jnp.int32, 16)
    cols = [(lanes + j) & 15 for j in range(16)]

    def xpose(src, dst):
        def kb_body(kb, carry):
            for kd in range(2):
                base = lanes32 + (16 * EMBED_DIM * kb + 16 * kd)
                for jh in range(2):
                    vals = [
                        plsc.load_gather(src, [base + cols[8 * jh + j]])
                        for j in range(8)
                    ]
                    for j in range(8):
                        plsc.store_scatter(
                            dst,
                            [16 * kd + cols[8 * jh + j], 16 * kb + lanes],
                            vals[j],
                        )
            return carry

        lax.fori_loop(0, 8, kb_body, 0)

    def write_out(t, buf, sem):
        q = q0 + t
        f = q // (BATCH // GRP)
        jb = q % (BATCH // GRP)
        pltpu.async_copy(
            buf,
            out_hbm.at[f, pl.ds(0, EMBED_DIM), pl.ds(jb * GRP, GRP)],
            sem,
        )

    HALF = PAIRS_PER_W // 2

    pltpu.sync_copy(x_hbm.at[pl.ds(q0 * PAIR_WORDS, PAIR_WORDS)], in0)
    start_in(1, in1, isem1)

    def step(u, carry):
        tA = 2 * u

        @pl.when(u > 0)
        def _():
            drain(in0, isem0)

        xpose(in0, ob0)

        @pl.when(u < HALF - 1)
        def _():
            start_in(tA + 2, in0, isem0)

        @pl.when(u > 0)
        def _():
            drain_ob(ob0, osem0)

        write_out(tA, ob0, osem0)

        drain(in1, isem1)
        xpose(in1, ob1)

        @pl.when(u < HALF - 1)
        def _():
            start_in(tA + 3, in1, isem1)

        @pl.when(u > 0)
        def _():
            drain_ob(ob1, osem1)

        write_out(tA + 1, ob1, osem1)
        return carry

    lax.fori_loop(0, HALF, step, 0)
    drain_ob(ob0, osem0)
    drain_ob(ob1, osem1)


def kernel(indices, table):
    fidx = indices.T.astype(jnp.int32).reshape(TOTAL // GRP, GRP)
    x = _sc_gather(fidx, table)
    out3 = _sc_out_xpose(x.reshape(TOTAL * EMBED_DIM))
    return jnp.transpose(out3, (2, 0, 1))
